# baseline (speedup 1.0000x reference)
import functools

import jax
import jax.numpy as jnp
from jax import lax
from jax.experimental import pallas as pl
from jax.experimental.pallas import tpu as pltpu
from jax.experimental.pallas import tpu_sc as plsc

B = 4
S = 2048
D = 768
N = B * S
NC, NS = 2, 16
NW = NC * NS
WPB = NW // B
TPW = N // NW
CH = 32
NCH = TPW // CH
NJ = D // 16

_mesh = plsc.VectorSubcoreMesh(core_axis_name="c", subcore_axis_name="s")


@functools.partial(
    pl.kernel,
    mesh=_mesh,
    out_type=jax.ShapeDtypeStruct((N, 16), jnp.float32),
    scratch_types=[
        pltpu.VMEM((NCH, CH), jnp.int32),
        pltpu.VMEM((CH, D), jnp.float32),
        pltpu.VMEM((CH, D), jnp.float32),
        pltpu.VMEM((D,), jnp.float32),
        pltpu.VMEM((TPW, 16), jnp.float32),
        pltpu.SemaphoreType.DMA,
        pltpu.SemaphoreType.DMA,
        pltpu.SemaphoreType.DMA,
        pltpu.SemaphoreType.DMA,
    ],
)
def _sc_score(x_hbm, table_hbm, w_hbm, s_hbm,
              idx_v, rows0, rows1, w_v, scores_v,
              gsem0, gsem1, ssem0, ssem1):
    wid = lax.axis_index("s") * NC + lax.axis_index("c")
    brow = wid // WPB
    scol = (wid % WPB) * TPW

    cp_is = [
        pltpu.async_copy(x_hbm.at[brow, pl.ds(scol + g * CH, CH)],
                         idx_v.at[g], ssem0)
        for g in range(NCH)
    ]
    cp_is[0].wait()

    rows = (rows0, rows1)
    gsems = (gsem0, gsem1)
    copies = [None, None]
    copies[0] = pltpu.async_copy(table_hbm.at[idx_v.at[0]], rows[0], gsems[0])

    cp_w = pltpu.async_copy(w_hbm, w_v, ssem1)
    for cp in cp_is[1:]:
        cp.wait()
    cp_w.wait()
    wv = tuple(w_v[pl.ds(j * 16, 16)] for j in range(NJ))

    for g in range(NCH):
        if g + 1 < NCH:
            nb = (g + 1) % 2
            copies[nb] = pltpu.async_copy(
                table_hbm.at[idx_v.at[g + 1]], rows[nb], gsems[nb])
        copies[g % 2].wait()
        rbuf = rows[g % 2]

        def body(r, carry):
            acc = [jnp.zeros((16,), jnp.float32) for _ in range(4)]
            for j in range(NJ):
                acc[j % 4] = acc[j % 4] + rbuf[r, pl.ds(j * 16, 16)] * wv[j]
            scores_v[g * CH + r, :] = (acc[0] + acc[1]) + (acc[2] + acc[3])
            return carry

        lax.fori_loop(0, CH, body, 0)

    pltpu.sync_copy(scores_v, s_hbm.at[pl.ds(wid * TPW, TPW)])


def _finish_body(s_ref, m_ref, o_ref):
    tok = jnp.sum(s_ref[...], axis=1).reshape(B, S)
    mf = m_ref[...].astype(jnp.float32)
    num = jnp.sum(tok * mf, axis=1)
    cnt = jnp.sum(mf, axis=1)
    o_ref[...] = num / jnp.clip(cnt, 1e-5, None)


def kernel(x, mask, embedding_table, prompt_embed, response_embed, W_pred):
    x_i = x if x.dtype == jnp.int32 else x.astype(jnp.int32)
    s = _sc_score(x_i, embedding_table, W_pred)
    pred = pl.pallas_call(
        _finish_body,
        out_shape=jax.ShapeDtypeStruct((B,), jnp.float32),
    )(s, mask)
    return pred


# speedup vs baseline: 1.2604x; 1.2604x over previous
import functools

import jax
import jax.numpy as jnp
from jax import lax
from jax.experimental import pallas as pl
from jax.experimental.pallas import tpu as pltpu
from jax.experimental.pallas import tpu_sc as plsc

B = 4
S = 2048
D = 768
N = B * S
NC, NS = 2, 16
NW = NC * NS
WPB = NW // B
TPW = N // NW
CH = 64
NCH = TPW // CH
NJ = D // 16

_mesh = plsc.VectorSubcoreMesh(core_axis_name="c", subcore_axis_name="s")


@functools.partial(
    pl.kernel,
    mesh=_mesh,
    out_type=jax.ShapeDtypeStruct((B, WPB * 16), jnp.float32),
    scratch_types=[
        pltpu.VMEM((NCH, CH), jnp.int32),
        pltpu.VMEM((CH, D), jnp.float32),
        pltpu.VMEM((CH, D), jnp.float32),
        pltpu.VMEM((D,), jnp.float32),
        pltpu.VMEM((16,), jnp.float32),
        pltpu.SemaphoreType.DMA,
        pltpu.SemaphoreType.DMA,
        pltpu.SemaphoreType.DMA,
        pltpu.SemaphoreType.DMA,
    ],
)
def _sc_pool(x_hbm, table_hbm, w_hbm, p_hbm,
             idx_v, rows0, rows1, w_v, pout,
             gsem0, gsem1, ssem0, ssem1):
    wid = lax.axis_index("s") * NC + lax.axis_index("c")
    brow = wid // WPB
    scol = (wid % WPB) * TPW

    cp_is = [
        pltpu.async_copy(x_hbm.at[brow, pl.ds(scol + g * CH, CH)],
                         idx_v.at[g], ssem0)
        for g in range(NCH)
    ]
    cp_is[0].wait()

    rows = (rows0, rows1)
    gsems = (gsem0, gsem1)
    copies = [None, None]
    copies[0] = pltpu.async_copy(table_hbm.at[idx_v.at[0]], rows[0], gsems[0])

    cp_w = pltpu.async_copy(w_hbm, w_v, ssem1)
    for cp in cp_is[1:]:
        cp.wait()

    accs = tuple(jnp.zeros((16,), jnp.float32) for _ in range(NJ))
    for g in range(NCH):
        if g + 1 < NCH:
            nb = (g + 1) % 2
            copies[nb] = pltpu.async_copy(
                table_hbm.at[idx_v.at[g + 1]], rows[nb], gsems[nb])
        copies[g % 2].wait()
        rbuf = rows[g % 2]

        def body(r, acc_t):
            return tuple(
                a + rbuf[r, pl.ds(j * 16, 16)] for j, a in enumerate(acc_t))

        accs = lax.fori_loop(0, CH, body, accs)

    cp_w.wait()

    dot = jnp.zeros((16,), jnp.float32)
    for j in range(NJ):
        dot = dot + accs[j] * w_v[pl.ds(j * 16, 16)]

    pout[...] = dot
    pltpu.sync_copy(pout, p_hbm.at[brow, pl.ds((wid % WPB) * 16, 16)])


def _finish_body(p_ref, m_ref, e0_ref, w_ref, o_ref):
    num = jnp.sum(p_ref[...], axis=1)
    cnt = jnp.sum(m_ref[...].astype(jnp.float32), axis=1)
    e0w = jnp.sum(e0_ref[...] * w_ref[...][None, :])
    num = num - (S - cnt) * e0w
    o_ref[...] = num / jnp.clip(cnt, 1e-5, None)


def kernel(x, mask, embedding_table, prompt_embed, response_embed, W_pred):
    xm = jnp.where(mask, x.astype(jnp.int32), 0)
    p = _sc_pool(xm, embedding_table, W_pred)
    pred = pl.pallas_call(
        _finish_body,
        out_shape=jax.ShapeDtypeStruct((B,), jnp.float32),
    )(p, mask, embedding_table[0:1], W_pred)
    return pred
